# baseline (speedup 1.0000x reference)
import functools

import jax
import jax.numpy as jnp
from jax import lax
from jax.experimental import pallas as pl
from jax.experimental.pallas import tpu as pltpu
from jax.experimental.pallas import tpu_sc as plsc

NUM_CORES = 2
NUM_SUBCORES = 16
LANES = 16
NUM_WORKERS = NUM_CORES * NUM_SUBCORES

BATCH = 16384
LATENT = 32
B_PER_W = BATCH // NUM_WORKERS
GATHER_CHUNK = 128
N_CHUNKS = B_PER_W // GATHER_CHUNK
GROUPS = B_PER_W // LANES


def _gmf_body(uidx_hbm, iidx_hbm, utab_hbm, itab_hbm, w_hbm, b_hbm, out_hbm,
              iu_v, ii_v, u_rows, i_rows, colbuf, out_v, wv, bv,
              sem_u, sem_i):
    wid = lax.axis_index("s") * NUM_CORES + lax.axis_index("c")
    base = wid * B_PER_W

    for c in range(N_CHUNKS):
        pltpu.sync_copy(uidx_hbm.at[pl.ds(base + c * GATHER_CHUNK, GATHER_CHUNK)],
                        iu_v.at[c])
        pltpu.sync_copy(iidx_hbm.at[pl.ds(base + c * GATHER_CHUNK, GATHER_CHUNK)],
                        ii_v.at[c])
    pltpu.sync_copy(w_hbm, wv)
    pltpu.sync_copy(b_hbm, bv)

    copies = []
    for c in range(N_CHUNKS):
        copies.append(pltpu.async_copy(
            utab_hbm.at[iu_v.at[c]],
            u_rows.at[pl.ds(c * GATHER_CHUNK, GATHER_CHUNK)], sem_u))
        copies.append(pltpu.async_copy(
            itab_hbm.at[ii_v.at[c]],
            i_rows.at[pl.ds(c * GATHER_CHUNK, GATHER_CHUNK)], sem_i))
    for cp in copies:
        cp.wait()

    iota = lax.iota(jnp.int32, LANES)
    iota_hi = iota + LANES
    w_lo = wv[pl.ds(0, LANES)]
    w_hi = wv[pl.ds(LANES, LANES)]
    b_vec = bv[...]

    iota16 = iota * LANES

    def group(g, carry):
        j0 = g * LANES
        for e in range(LANES):
            j = j0 + e
            u_lo = u_rows[j, pl.ds(0, LANES)]
            u_hi = u_rows[j, pl.ds(LANES, LANES)]
            i_lo = i_rows[j, pl.ds(0, LANES)]
            i_hi = i_rows[j, pl.ds(LANES, LANES)]
            p = u_lo * i_lo * w_lo + u_hi * i_hi * w_hi
            plsc.store_scatter(colbuf, [iota16 + e], p)
        acc = colbuf[pl.ds(0, LANES)]
        for r in range(1, LANES):
            acc = acc + colbuf[pl.ds(r * LANES, LANES)]
        t = acc + b_vec
        sig = 1.0 / (1.0 + jnp.exp(-t))
        plsc.store_scatter(out_v, [j0 + iota], sig)
        return carry

    lax.fori_loop(0, GROUPS, group, 0)

    pltpu.sync_copy(out_v, out_hbm.at[pl.ds(base, B_PER_W)])


@jax.jit
def _gmf(user_indices, item_indices, user_table, item_table, w_flat, b_vec):
    mesh = plsc.VectorSubcoreMesh(core_axis_name="c", subcore_axis_name="s",
                                  num_cores=NUM_CORES, num_subcores=NUM_SUBCORES)
    run = pl.kernel(
        _gmf_body,
        out_type=jax.ShapeDtypeStruct((BATCH,), jnp.float32),
        mesh=mesh,
        compiler_params=pltpu.CompilerParams(needs_layout_passes=False,
                                             use_tc_tiling_on_sc=False),
        scratch_types=[
            pltpu.VMEM((N_CHUNKS, GATHER_CHUNK), jnp.int32),
            pltpu.VMEM((N_CHUNKS, GATHER_CHUNK), jnp.int32),
            pltpu.VMEM((B_PER_W, LATENT), jnp.float32),
            pltpu.VMEM((B_PER_W, LATENT), jnp.float32),
            pltpu.VMEM((LANES * LANES,), jnp.float32),
            pltpu.VMEM((B_PER_W,), jnp.float32),
            pltpu.VMEM((LATENT,), jnp.float32),
            pltpu.VMEM((LANES,), jnp.float32),
            pltpu.SemaphoreType.DMA,
            pltpu.SemaphoreType.DMA,
        ],
    )
    return run(user_indices, item_indices, user_table, item_table, w_flat, b_vec)


def kernel(user_indices, item_indices, user_table, item_table, W, b):
    w_flat = W.reshape(LATENT)
    b_vec = jnp.broadcast_to(b, (LANES,))
    out = _gmf(user_indices.astype(jnp.int32), item_indices.astype(jnp.int32),
               user_table, item_table, w_flat, b_vec)
    return out.reshape(BATCH, 1)
